# baseline (speedup 1.0000x reference)
import functools

import jax
import jax.numpy as jnp
from jax.experimental import pallas as pl


def _pe_kernel(ids_ref, emb_ref, pos_ref, out_ref, mask_ref, *, scale):
    m = (ids_ref[...] != 0)
    mf = m.astype(jnp.float32)
    out_ref[...] = (emb_ref[...] * scale + pos_ref[...][None, :, :]) * mf[:, :, None]
    mask_ref[...] = mf


def kernel(past_ids, past_embeddings, pos_table):
    b, n = past_ids.shape
    d = past_embeddings.shape[-1]
    scale = float(d) ** 0.5
    bb = 32
    grid = (b // bb,)
    out, mask2d = pl.pallas_call(
        functools.partial(_pe_kernel, scale=scale),
        grid=grid,
        in_specs=[
            pl.BlockSpec((bb, n), lambda i: (i, 0)),
            pl.BlockSpec((bb, n, d), lambda i: (i, 0, 0)),
            pl.BlockSpec((n, d), lambda i: (0, 0)),
        ],
        out_specs=[
            pl.BlockSpec((bb, n, d), lambda i: (i, 0, 0)),
            pl.BlockSpec((bb, n), lambda i: (i, 0)),
        ],
        out_shape=[
            jax.ShapeDtypeStruct((b, n, d), jnp.float32),
            jax.ShapeDtypeStruct((b, n), jnp.float32),
        ],
    )(past_ids, past_embeddings, pos_table)
    return (out, mask2d[:, :, None])
